# baseline (device time: 43453 ns/iter reference)
import jax
import jax.numpy as jnp
from jax import lax
from jax.experimental import pallas as pl
from jax.experimental.pallas import tpu as pltpu

N_DEV = 4
Q = 8


def kernel(x):
    m, n = x.shape
    m2 = m // 2
    mh = m // 4
    qh = mh // Q

    def body(x_ref, out_ref, comm_ref, send_sems, recv_sems):
        my = lax.axis_index("i")
        p1 = my + 1 - 2 * lax.rem(my, 2)
        p2 = 3 - my

        barrier_sem = pltpu.get_barrier_semaphore()
        for nbr in (p1, p2):
            pl.semaphore_signal(
                barrier_sem, inc=1,
                device_id=(nbr,), device_id_type=pl.DeviceIdType.MESH,
            )
        pl.semaphore_wait(barrier_sem, 2)

        h_a = jnp.where(jnp.logical_or(my == 0, my == 3), 0, mh)
        h_b = jnp.where(my < 2, 0, mh)
        own = [0 * m2 + h_a, 1 * m2 + h_b]
        oth = [0 * m2 + (mh - h_a), 1 * m2 + (mh - h_b)]
        partners = [(p1, p2, p1), (p2, p1, p2)]

        def sem_idx(b, stage, q):
            return (b * 3 + stage) * Q + q

        def exchange(b, stage, q, src_ref, dst_ref):
            return pltpu.make_async_remote_copy(
                src_ref=src_ref,
                dst_ref=dst_ref,
                send_sem=send_sems.at[sem_idx(b, stage, q)],
                recv_sem=recv_sems.at[sem_idx(b, stage, q)],
                device_id=(partners[b][stage],),
                device_id_type=pl.DeviceIdType.MESH,
            )

        s1 = [[None] * Q, [None] * Q]
        for q in range(Q):
            for b in range(2):
                r = exchange(
                    b, 0, q,
                    x_ref.at[pl.ds(oth[b] + q * qh, qh), :],
                    comm_ref.at[2 * b, pl.ds(q * qh, qh), :],
                )
                r.start()
                s1[b][q] = r

        s2 = [[None] * Q, [None] * Q]
        for q in range(Q):
            for b in range(2):
                s1[b][q].wait()
                rows = pl.ds(own[b] + q * qh, qh)
                crows = pl.ds(q * qh, qh)
                out_ref[rows, :] = x_ref[rows, :] + comm_ref[2 * b, crows, :]
                r = exchange(b, 1, q, out_ref.at[rows, :],
                             comm_ref.at[2 * b + 1, crows, :])
                r.start()
                s2[b][q] = r

        s3 = [[None] * Q, [None] * Q]
        for q in range(Q):
            for b in range(2):
                s2[b][q].wait()
                rows = pl.ds(own[b] + q * qh, qh)
                crows = pl.ds(q * qh, qh)
                out_ref[rows, :] = out_ref[rows, :] + comm_ref[2 * b + 1, crows, :]
                r = exchange(b, 2, q, out_ref.at[rows, :], out_ref.at[rows, :])
                r.start()
                s3[b][q] = r

        for q in range(Q):
            for b in range(2):
                s3[b][q].wait()

    out_shape = jax.ShapeDtypeStruct((m, n), x.dtype)
    return pl.pallas_call(
        body,
        out_shape=out_shape,
        in_specs=[pl.BlockSpec(memory_space=pltpu.VMEM)],
        out_specs=pl.BlockSpec(memory_space=pltpu.VMEM),
        scratch_shapes=[
            pltpu.VMEM((4, mh, n), x.dtype),
            pltpu.SemaphoreType.DMA((6 * Q,)),
            pltpu.SemaphoreType.DMA((6 * Q,)),
        ],
        compiler_params=pltpu.CompilerParams(collective_id=0),
    )(x)


# device time: 43319 ns/iter; 1.0031x vs baseline; 1.0031x over previous
import jax
import jax.numpy as jnp
from jax import lax
from jax.experimental import pallas as pl
from jax.experimental.pallas import tpu as pltpu

N_DEV = 4
Q = 4


def kernel(x):
    m, n = x.shape
    m2 = m // 2
    mh = m // 4
    qh = mh // Q

    def body(x_ref, out_ref, acc_ref, comm1_ref, comm2_ref,
             send_sems, recv_sems, load_sems, store_sems):
        my = lax.axis_index("i")
        p1 = my + 1 - 2 * lax.rem(my, 2)
        p2 = 3 - my

        h_a = jnp.where(jnp.logical_or(my == 0, my == 3), 0, mh)
        h_b = jnp.where(my < 2, 0, mh)
        own = [0 * m2 + h_a, 1 * m2 + h_b]
        oth = [0 * m2 + (mh - h_a), 1 * m2 + (mh - h_b)]
        partners = [(p1, p2, p1), (p2, p1, p2)]

        loads = [[None] * Q, [None] * Q]
        for q in range(Q):
            for b in range(2):
                cp = pltpu.make_async_copy(
                    x_ref.at[pl.ds(own[b] + q * qh, qh), :],
                    acc_ref.at[b, pl.ds(q * qh, qh), :],
                    load_sems.at[b * Q + q],
                )
                cp.start()
                loads[b][q] = cp

        barrier_sem = pltpu.get_barrier_semaphore()
        for nbr in (p1, p2):
            pl.semaphore_signal(
                barrier_sem, inc=1,
                device_id=(nbr,), device_id_type=pl.DeviceIdType.MESH,
            )
        pl.semaphore_wait(barrier_sem, 2)

        def exchange(b, stage, q, src_ref, dst_ref):
            return pltpu.make_async_remote_copy(
                src_ref=src_ref,
                dst_ref=dst_ref,
                send_sem=send_sems.at[(b * 3 + stage) * Q + q],
                recv_sem=recv_sems.at[(b * 3 + stage) * Q + q],
                device_id=(partners[b][stage],),
                device_id_type=pl.DeviceIdType.MESH,
            )

        s1 = [[None] * Q, [None] * Q]
        for q in range(Q):
            for b in range(2):
                r = exchange(
                    b, 0, q,
                    x_ref.at[pl.ds(oth[b] + q * qh, qh), :],
                    comm1_ref.at[b, pl.ds(q * qh, qh), :],
                )
                r.start()
                s1[b][q] = r

        s2 = [[None] * Q, [None] * Q]
        for q in range(Q):
            for b in range(2):
                s1[b][q].wait()
                loads[b][q].wait()
                crows = pl.ds(q * qh, qh)
                acc_ref[b, crows, :] = acc_ref[b, crows, :] + comm1_ref[b, crows, :]
                r = exchange(b, 1, q, acc_ref.at[b, crows, :],
                             comm2_ref.at[b, crows, :])
                r.start()
                s2[b][q] = r

        s3 = [[None] * Q, [None] * Q]
        stores = [[None] * Q, [None] * Q]
        for q in range(Q):
            for b in range(2):
                s2[b][q].wait()
                crows = pl.ds(q * qh, qh)
                rows = pl.ds(own[b] + q * qh, qh)
                acc_ref[b, crows, :] = acc_ref[b, crows, :] + comm2_ref[b, crows, :]
                r = exchange(b, 2, q, acc_ref.at[b, crows, :],
                             out_ref.at[rows, :])
                r.start()
                s3[b][q] = r
                cp = pltpu.make_async_copy(
                    acc_ref.at[b, crows, :],
                    out_ref.at[rows, :],
                    store_sems.at[b * Q + q],
                )
                cp.start()
                stores[b][q] = cp

        for q in range(Q):
            for b in range(2):
                s3[b][q].wait()
                stores[b][q].wait()

    out_shape = jax.ShapeDtypeStruct((m, n), x.dtype)
    return pl.pallas_call(
        body,
        out_shape=out_shape,
        in_specs=[pl.BlockSpec(memory_space=pl.ANY)],
        out_specs=pl.BlockSpec(memory_space=pl.ANY),
        scratch_shapes=[
            pltpu.VMEM((2, mh, n), x.dtype),
            pltpu.VMEM((2, mh, n), x.dtype),
            pltpu.VMEM((2, mh, n), x.dtype),
            pltpu.SemaphoreType.DMA((6 * Q,)),
            pltpu.SemaphoreType.DMA((6 * Q,)),
            pltpu.SemaphoreType.DMA((2 * Q,)),
            pltpu.SemaphoreType.DMA((2 * Q,)),
        ],
        compiler_params=pltpu.CompilerParams(collective_id=0),
    )(x)
